# baseline (speedup 1.0000x reference)
import jax
import jax.numpy as jnp
from jax.experimental import pallas as pl
from jax.experimental.pallas import tpu as pltpu

_N = 5000
_NPAD = 5120
_DRUG = 2000
_TARGET = 3000
_ALPHA = 0.1

_BM = 512
_BM1 = 1024
_BMD = 400
_NS = -(-_N // _BM)
_NS1 = _NPAD // _BM1
_ND = _DRUG // _BMD
_P1 = _NS
_P2 = _NS + _NS1
_NSTEPS = _P2 + _ND


def _fused_kernel(h_ref, g_ref, w1_ref, b1_ref, w2_ref, b2_ref,
                  l1w_ref, l1b_ref, l2w_ref, l2b_ref, tw_ref, w_ref,
                  o_ref, a_scr, b_scr, g8_scr, hc_scr):
    i = pl.program_id(0)
    f32 = jnp.float32
    bf16 = jnp.bfloat16

    @pl.when(i == 0)
    def _():
        a_scr[...] = jnp.dot(h_ref[...], w1_ref[...],
                             preferred_element_type=f32).astype(bf16)

    @pl.when(i < _P1)
    def _():
        gf = g_ref[...]
        g = gf.astype(bf16)
        h = jnp.dot(g, a_scr[...],
                    preferred_element_type=f32) + b1_ref[...]
        h = jnp.where(h > 0, h, 0.25 * h)
        b_scr[pl.ds(i * _BM, _BM), :] = jnp.dot(
            h, w2_ref[...], preferred_element_type=f32).astype(bf16)
        g8_scr[pl.ds(i * _BM, _BM), :] = gf.astype(jnp.int8)

    is_p1 = jnp.logical_or(jnp.logical_and(i >= _P1, i < _P1 + 4),
                           i == _P1 + 6)
    is_dec = jnp.logical_and(i >= _P1 + 4, jnp.logical_not(is_p1))

    @pl.when(is_p1)
    def _():
        row = jnp.where(i < _P1 + 4, (i - (_P1 - 1)) * _BM1, 0)
        row = pl.multiple_of(row, 1024)
        g = g8_scr[pl.ds(row, _BM1), :].astype(bf16)
        b = b_scr[:_N, :]
        h = jnp.dot(g, b, preferred_element_type=f32) + b2_ref[...]
        h1 = jnp.where(h > 0, h, 0.25 * h)
        x = jnp.dot(h1, l1w_ref[...], preferred_element_type=f32) + l1b_ref[...]
        x = jnp.maximum(x, 0.0)
        x = jnp.dot(x, l2w_ref[...], preferred_element_type=f32) + l2b_ref[...]
        w = w_ref[0, 0]
        hc_scr[pl.ds(row, _BM1), :] = (
            w * h1 + (1.0 - w) * _ALPHA * x).astype(bf16)

    @pl.when(is_dec)
    def _():
        k = jnp.where(i <= _P1 + 5, i - (_P1 + 1), i - (_P1 + 7))
        hr = hc_scr[pl.ds(pl.multiple_of(k * _BMD, 8), _BMD), :]
        u = jnp.dot(hr, tw_ref[...].astype(bf16),
                    preferred_element_type=f32).astype(bf16)
        hd = hc_scr[_DRUG:_N, :]
        o_ref[...] = jax.lax.dot_general(
            u, hd, (((1,), (1,)), ((), ())), preferred_element_type=f32)


def kernel(H, G, W1, b1, W2, b2, l1W, l1b, l2W, l2b, train_W,
           drug_num, target_num, w):
    f32 = jnp.float32
    b1r = b1.reshape(1, -1).astype(f32)
    b2r = b2.reshape(1, -1).astype(f32)
    l1br = l1b.reshape(1, -1).astype(f32)
    l2br = l2b.reshape(1, -1).astype(f32)
    w_arr = jnp.asarray(w, f32).reshape(1, 1)

    hgcn = W1.shape[1]
    hidden = l1W.shape[1]

    full = lambda i: (0, 0)
    out = pl.pallas_call(
        _fused_kernel,
        grid=(_NSTEPS,),
        in_specs=[
            pl.BlockSpec((_N, hgcn), full),
            pl.BlockSpec((_BM, _N), lambda i: (jnp.minimum(i, _NS - 1), 0)),
            pl.BlockSpec((hgcn, hgcn), full),
            pl.BlockSpec((1, hgcn), full),
            pl.BlockSpec((hgcn, hgcn), full),
            pl.BlockSpec((1, hgcn), full),
            pl.BlockSpec((hgcn, hidden), full),
            pl.BlockSpec((1, hidden), full),
            pl.BlockSpec((hidden, hgcn), full),
            pl.BlockSpec((1, hgcn), full),
            pl.BlockSpec((hgcn, hgcn), full),
            pl.BlockSpec((1, 1), full),
        ],
        out_specs=pl.BlockSpec(
            (_BMD, _TARGET),
            lambda i: (jnp.where(i <= _P1 + 4, 3,
                                 jnp.where(i <= _P1 + 6, 4, i - (_P1 + 7))),
                       0)),
        out_shape=jax.ShapeDtypeStruct((_DRUG, _TARGET), f32),
        scratch_shapes=[
            pltpu.VMEM((_N, hgcn), jnp.bfloat16),
            pltpu.VMEM((_NPAD, hgcn), jnp.bfloat16),
            pltpu.VMEM((_NPAD, _N), jnp.int8),
            pltpu.VMEM((_NPAD, hgcn), jnp.bfloat16),
        ],
        compiler_params=pltpu.CompilerParams(
            vmem_limit_bytes=100 * 1024 * 1024),
    )(H, G, W1, b1r, W2, b2r, l1W, l1br, l2W, l2br, train_W, w_arr)

    return out


# speedup vs baseline: 1.0051x; 1.0051x over previous
import jax
import jax.numpy as jnp
from jax.experimental import pallas as pl
from jax.experimental.pallas import tpu as pltpu

_N = 5000
_NPAD = 5120
_DRUG = 2000
_TARGET = 3000
_ALPHA = 0.1

_BM = 512
_BM1 = 1024
_BMD = 400
_NS = -(-_N // _BM)
_NS1 = _NPAD // _BM1
_ND = _DRUG // _BMD
_P1 = _NS
_P2 = _NS + _NS1
_NSTEPS = _P2 + _ND


def _fused_kernel(h_ref, g_ref, w1_ref, b1_ref, w2_ref, b2_ref,
                  l1w_ref, l1b_ref, l2w_ref, l2b_ref, tw_ref, w_ref,
                  o_ref, a_scr, b_scr, g8_scr, hc_scr):
    i = pl.program_id(0)
    f32 = jnp.float32
    bf16 = jnp.bfloat16

    @pl.when(i == 0)
    def _():
        a_scr[...] = jnp.dot(h_ref[...], w1_ref[...],
                             preferred_element_type=f32).astype(bf16)

    @pl.when(i < _P1)
    def _():
        gf = g_ref[...]
        g = gf.astype(bf16)
        h = jnp.dot(g, a_scr[...],
                    preferred_element_type=f32) + b1_ref[...]
        h = jnp.where(h > 0, h, 0.25 * h)
        b_scr[pl.ds(i * _BM, _BM), :] = jnp.dot(
            h, w2_ref[...], preferred_element_type=f32).astype(bf16)
        g8_scr[pl.ds(i * _BM, _BM), :] = gf.astype(jnp.int8)

    @pl.when(jnp.logical_and(i >= _P1, i < _P2))
    def _():
        j = i - _P1
        g = g8_scr[pl.ds(j * _BM1, _BM1), :].astype(bf16)
        b = b_scr[:_N, :]
        h = jnp.dot(g, b, preferred_element_type=f32) + b2_ref[...]
        h1 = jnp.where(h > 0, h, 0.25 * h)
        x = jnp.dot(h1, l1w_ref[...], preferred_element_type=f32) + l1b_ref[...]
        x = jnp.maximum(x, 0.0)
        x = jnp.dot(x, l2w_ref[...], preferred_element_type=f32) + l2b_ref[...]
        w = w_ref[0, 0]
        hc_scr[pl.ds(j * _BM1, _BM1), :] = (
            w * h1 + (1.0 - w) * _ALPHA * x).astype(bf16)

    @pl.when(i >= _P2)
    def _():
        k = i - _P2
        hr = hc_scr[pl.ds(k * _BMD, _BMD), :]
        u = jnp.dot(hr, tw_ref[...].astype(bf16),
                    preferred_element_type=f32).astype(bf16)
        hd = hc_scr[_DRUG:_N, :]
        o_ref[...] = jax.lax.dot_general(
            u, hd, (((1,), (1,)), ((), ())), preferred_element_type=f32)


def kernel(H, G, W1, b1, W2, b2, l1W, l1b, l2W, l2b, train_W,
           drug_num, target_num, w):
    f32 = jnp.float32
    b1r = b1.reshape(1, -1).astype(f32)
    b2r = b2.reshape(1, -1).astype(f32)
    l1br = l1b.reshape(1, -1).astype(f32)
    l2br = l2b.reshape(1, -1).astype(f32)
    w_arr = jnp.asarray(w, f32).reshape(1, 1)

    hgcn = W1.shape[1]
    hidden = l1W.shape[1]

    full = lambda i: (0, 0)
    out = pl.pallas_call(
        _fused_kernel,
        grid=(_NSTEPS,),
        in_specs=[
            pl.BlockSpec((_N, hgcn), full),
            pl.BlockSpec((_BM, _N), lambda i: (jnp.minimum(i, _NS - 1), 0)),
            pl.BlockSpec((hgcn, hgcn), full),
            pl.BlockSpec((1, hgcn), full),
            pl.BlockSpec((hgcn, hgcn), full),
            pl.BlockSpec((1, hgcn), full),
            pl.BlockSpec((hgcn, hidden), full),
            pl.BlockSpec((1, hidden), full),
            pl.BlockSpec((hidden, hgcn), full),
            pl.BlockSpec((1, hgcn), full),
            pl.BlockSpec((hgcn, hgcn), full),
            pl.BlockSpec((1, 1), full),
        ],
        out_specs=pl.BlockSpec(
            (_BMD, _TARGET), lambda i: (jnp.maximum(i - _P2, 0), 0)),
        out_shape=jax.ShapeDtypeStruct((_DRUG, _TARGET), f32),
        scratch_shapes=[
            pltpu.VMEM((_N, hgcn), jnp.bfloat16),
            pltpu.VMEM((_NPAD, hgcn), jnp.bfloat16),
            pltpu.VMEM((_NPAD, _N), jnp.int8),
            pltpu.VMEM((_NPAD, hgcn), jnp.bfloat16),
        ],
        compiler_params=pltpu.CompilerParams(
            vmem_limit_bytes=100 * 1024 * 1024),
    )(H, G, W1, b1r, W2, b2r, l1W, l1br, l2W, l2br, train_W, w_arr)

    return out
